# baseline (speedup 1.0000x reference)
import functools

import jax
import jax.numpy as jnp
from jax import lax
from jax.experimental import pallas as pl
from jax.experimental.pallas import tpu as pltpu
from jax.experimental.pallas import tpu_sc as plsc

_CARDS = 100000
_D = 64
_CTX = 200
_HID = 128
_IN1 = _CTX * _D

_NW = 32
_ROWS_PER_W = 8
_PAD_B = _NW * _ROWS_PER_W
_PAD_IDX = _PAD_B + 8

_BV = 8192
_NCH = _CARDS // _BV
_TAIL_OFF = _NCH * _BV
_TAIL_W = _CARDS - _TAIL_OFF
_NBUF = 3

_sc_mesh = plsc.VectorSubcoreMesh(core_axis_name="c", subcore_axis_name="s")


@functools.partial(
    pl.kernel,
    mesh=_sc_mesh,
    out_type=jax.ShapeDtypeStruct((_PAD_B, _D), jnp.float32),
    scratch_types=[
        pltpu.VMEM((16,), jnp.int32),
        pltpu.VMEM((_ROWS_PER_W, _D), jnp.float32),
        pltpu.SemaphoreType.DMA,
    ],
)
def _sc_gather(table_hbm, idx_hbm, out_hbm, idx_v, rows_v, sem):
    wid = lax.axis_index("s") * 2 + lax.axis_index("c")
    base = wid * _ROWS_PER_W
    pltpu.sync_copy(idx_hbm.at[pl.ds(base, 16)], idx_v)
    idx = idx_v[...]
    copies = []
    for i in range(_ROWS_PER_W):
        copies.append(
            pltpu.async_copy(
                table_hbm.at[pl.ds(idx[i], 1)], rows_v.at[pl.ds(i, 1)], sem
            )
        )
    for c in copies:
        c.wait()
    pltpu.sync_copy(rows_v, out_hbm.at[pl.ds(base, _ROWS_PER_W)])


def _fused_body(e_ref, w1_ref, b1_ref, b2_ref, w2tail_ref, w2_hbm, out_ref, *rest):
    bufs = rest[:_NBUF]
    flat_sems = rest[_NBUF:]
    sems = [(flat_sems[2 * i], flat_sems[2 * i + 1]) for i in range(_NBUF)]
    copies = [None] * _NCH

    def start(j):
        b = bufs[j % _NBUF]
        c0 = pltpu.make_async_copy(
            w2_hbm.at[pl.ds(0, 64), pl.ds(j * _BV, _BV)],
            b.at[pl.ds(0, 64), :],
            sems[j % _NBUF][0],
        )
        c1 = pltpu.make_async_copy(
            w2_hbm.at[pl.ds(64, 64), pl.ds(j * _BV, _BV)],
            b.at[pl.ds(64, 64), :],
            sems[j % _NBUF][1],
        )
        c0.start()
        c1.start()
        copies[j] = (c0, c1)

    for j in range(min(_NBUF, _NCH)):
        start(j)

    h = jnp.dot(e_ref[...], w1_ref[...], preferred_element_type=jnp.float32)
    h = jnp.maximum(h + b1_ref[...], 0.0)

    zt = jnp.dot(h, w2tail_ref[...], preferred_element_type=jnp.float32)
    zt = zt + b2_ref[:, _TAIL_OFF:]
    out_ref[:, _TAIL_OFF:] = zt
    m = jnp.max(zt)
    s = jnp.sum(jnp.exp(zt - m))

    for j in range(_NCH):
        copies[j][0].wait()
        copies[j][1].wait()
        if j + _NBUF < _NCH:
            start(j + _NBUF)
        off = j * _BV
        z = jnp.dot(h, bufs[j % _NBUF][...], preferred_element_type=jnp.float32)
        z = z + b2_ref[:, off : off + _BV]
        out_ref[:, off : off + _BV] = z
        bm = jnp.max(z)
        mn = jnp.maximum(m, bm)
        s = s * jnp.exp(m - mn) + jnp.sum(jnp.exp(z - mn))
        m = mn
    lse = m + jnp.log(s)
    out_ref[...] = out_ref[...] - lse


def kernel(inputs, emb_table, W1, b1, W2, b2):
    idx = jnp.zeros((_PAD_IDX,), jnp.int32).at[:_CTX].set(inputs)
    rows = _sc_gather(emb_table, idx)
    e = rows[:_CTX].reshape(1, _IN1)

    log_probs = pl.pallas_call(
        _fused_body,
        in_specs=[
            pl.BlockSpec(memory_space=pltpu.VMEM),
            pl.BlockSpec(memory_space=pltpu.VMEM),
            pl.BlockSpec(memory_space=pltpu.VMEM),
            pl.BlockSpec(memory_space=pltpu.VMEM),
            pl.BlockSpec(memory_space=pltpu.VMEM),
            pl.BlockSpec(memory_space=pltpu.MemorySpace.HBM),
        ],
        out_specs=pl.BlockSpec(memory_space=pltpu.VMEM),
        out_shape=jax.ShapeDtypeStruct((1, _CARDS), jnp.float32),
        scratch_shapes=(
            [pltpu.VMEM((_HID, _BV), jnp.float32) for _ in range(_NBUF)]
            + [pltpu.SemaphoreType.DMA for _ in range(2 * _NBUF)]
        ),
    )(
        e,
        W1,
        b1.reshape(1, _HID),
        b2.reshape(1, _CARDS),
        lax.slice(W2, (0, _TAIL_OFF), (_HID, _CARDS)),
        W2,
    )

    return log_probs


# speedup vs baseline: 1.4719x; 1.4719x over previous
import jax
import jax.numpy as jnp
from jax import lax
from jax.experimental import pallas as pl
from jax.experimental.pallas import tpu as pltpu

_CARDS = 100000
_D = 64
_CTX = 200
_HID = 128
_IN1 = _CTX * _D

_BV = 8192
_NCH = _CARDS // _BV
_TAIL_OFF = _NCH * _BV
_TAIL_W = _CARDS - _TAIL_OFF
_NBUF = 3


_NGB = 8


def _body(idx_ref, b1_ref, b2_ref, tableT_hbm, w1_hbm, w2t_hbm, out_ref, *rest):
    w1v, tailbuf = rest[0], rest[1]
    bufs = rest[2 : 2 + _NBUF]
    gbufs = rest[2 + _NBUF : 2 + _NBUF + _NGB]
    w2sems = rest[2 + _NBUF + _NGB : 2 + 2 * _NBUF + _NGB]
    gsems = rest[2 + 2 * _NBUF + _NGB : 2 + 2 * _NBUF + 2 * _NGB]
    w1sem, tailsem = rest[2 + 2 * _NBUF + 2 * _NGB :]

    copies = [None] * _NCH

    def start(j):
        c = pltpu.make_async_copy(
            w2t_hbm.at[pl.ds(j * _BV, _BV), :],
            bufs[j % _NBUF],
            w2sems[j % _NBUF],
        )
        c.start()
        copies[j] = c

    w1_copy = pltpu.make_async_copy(w1_hbm, w1v, w1sem)
    w1_copy.start()

    def gstart(t):
        a = idx_ref[t]
        base = pl.multiple_of((a // 128) * 128, 128)
        c = pltpu.make_async_copy(
            tableT_hbm.at[:, pl.ds(base, 128)],
            gbufs[t % _NGB],
            gsems[t % _NGB],
        )
        c.start()
        return c

    gcopies = [None] * _CTX
    for t in range(_NGB):
        gcopies[t] = gstart(t)

    for j in range(_NBUF):
        start(j)
    tail_copy = pltpu.make_async_copy(
        w2t_hbm.at[pl.ds(_TAIL_OFF, _TAIL_W), :], tailbuf, tailsem
    )
    tail_copy.start()

    w1_copy.wait()

    sub_iota = lax.broadcasted_iota(jnp.int32, (128, 1), 0)
    accs = [jnp.zeros((1, _HID), jnp.float32) for _ in range(4)]
    for t in range(_CTX):
        gcopies[t].wait()
        a = idx_ref[t]
        oh = jnp.where(sub_iota == a % 128, 1.0, 0.0).astype(jnp.float32)
        col = lax.dot_general(
            gbufs[t % _NGB][...], oh, (((1,), (0,)), ((), ())),
            preferred_element_type=jnp.float32,
        )
        if t + _NGB < _CTX:
            gcopies[t + _NGB] = gstart(t + _NGB)
        p = lax.dot_general(
            col,
            w1v[_D * t : _D * (t + 1), :],
            (((0,), (0,)), ((), ())),
            preferred_element_type=jnp.float32,
        )
        accs[t % 4] = accs[t % 4] + p
    h = accs[0] + accs[1] + accs[2] + accs[3] + b1_ref[...]
    h = jnp.maximum(h, 0.0)

    tail_copy.wait()
    zt = lax.dot_general(
        h, tailbuf[...], (((1,), (1,)), ((), ())),
        preferred_element_type=jnp.float32,
    )
    zt = zt + b2_ref[:, _TAIL_OFF:]
    out_ref[:, _TAIL_OFF:] = zt
    m = jnp.max(zt)
    s = jnp.sum(jnp.exp(zt - m))

    for j in range(_NCH):
        copies[j].wait()
        if j + _NBUF < _NCH:
            start(j + _NBUF)
        off = j * _BV
        z = lax.dot_general(
            h, bufs[j % _NBUF][...], (((1,), (1,)), ((), ())),
            preferred_element_type=jnp.float32,
        )
        z = z + b2_ref[:, off : off + _BV]
        out_ref[:, off : off + _BV] = z
        bm = jnp.max(z)
        mn = jnp.maximum(m, bm)
        s = s * jnp.exp(m - mn) + jnp.sum(jnp.exp(z - mn))
        m = mn

    lse = m + jnp.log(s)
    out_ref[...] = out_ref[...] - lse


def kernel(inputs, emb_table, W1, b1, W2, b2):
    tableT = emb_table.T
    w2t = W2.T

    log_probs = pl.pallas_call(
        _body,
        in_specs=[
            pl.BlockSpec(memory_space=pltpu.SMEM),
            pl.BlockSpec(memory_space=pltpu.VMEM),
            pl.BlockSpec(memory_space=pltpu.VMEM),
            pl.BlockSpec(memory_space=pltpu.MemorySpace.HBM),
            pl.BlockSpec(memory_space=pltpu.MemorySpace.HBM),
            pl.BlockSpec(memory_space=pltpu.MemorySpace.HBM),
        ],
        out_specs=pl.BlockSpec(memory_space=pltpu.VMEM),
        out_shape=jax.ShapeDtypeStruct((1, _CARDS), jnp.float32),
        scratch_shapes=(
            [
                pltpu.VMEM((_IN1, _HID), jnp.float32),
                pltpu.VMEM((_TAIL_W, _HID), jnp.float32),
            ]
            + [pltpu.VMEM((_BV, _HID), jnp.float32) for _ in range(_NBUF)]
            + [pltpu.VMEM((_D, 128), jnp.float32) for _ in range(_NGB)]
            + [pltpu.SemaphoreType.DMA for _ in range(_NBUF + _NGB + 2)]
        ),
    )(inputs, b1.reshape(1, _HID), b2.reshape(1, _CARDS), tableT, W1, w2t)

    return log_probs


# speedup vs baseline: 1.8920x; 1.2854x over previous
import jax
import jax.numpy as jnp
from jax import lax
from jax.experimental import pallas as pl
from jax.experimental.pallas import tpu as pltpu

_CARDS = 100000
_D = 64
_CTX = 200
_HID = 128
_IN1 = _CTX * _D

_BV = 8192
_NCH = _CARDS // _BV
_TAIL_OFF = _NCH * _BV
_TAIL_W = _CARDS - _TAIL_OFF
_NBUF = 3


_NGB = 16


def _body(idx_ref, b1_ref, b2_ref, tableT_hbm, w1_hbm, w2t_hbm, out_ref, *rest):
    w1v, tailbuf, eflat = rest[0], rest[1], rest[2]
    bufs = rest[3 : 3 + _NBUF]
    gbufs = rest[3 + _NBUF : 3 + _NBUF + _NGB]
    w2sems = rest[3 + _NBUF + _NGB : 3 + 2 * _NBUF + _NGB]
    gsems = rest[3 + 2 * _NBUF + _NGB : 3 + 2 * _NBUF + 2 * _NGB]
    w1sem, tailsem = rest[3 + 2 * _NBUF + 2 * _NGB :]

    copies = [None] * _NCH

    def start(j):
        c = pltpu.make_async_copy(
            w2t_hbm.at[pl.ds(j * _BV, _BV), :],
            bufs[j % _NBUF],
            w2sems[j % _NBUF],
        )
        c.start()
        copies[j] = c

    w1_copy = pltpu.make_async_copy(w1_hbm, w1v, w1sem)
    w1_copy.start()

    def gstart(t):
        a = idx_ref[t]
        base = pl.multiple_of((a // 128) * 128, 128)
        c = pltpu.make_async_copy(
            tableT_hbm.at[:, pl.ds(base, 128)],
            gbufs[t % _NGB],
            gsems[t % _NGB],
        )
        c.start()
        return c

    gcopies = [None] * _CTX
    for t in range(_NGB):
        gcopies[t] = gstart(t)

    for j in range(_NBUF):
        start(j)
    tail_copy = pltpu.make_async_copy(
        w2t_hbm.at[pl.ds(_TAIL_OFF, _TAIL_W), :], tailbuf, tailsem
    )
    tail_copy.start()

    sub_iota = lax.broadcasted_iota(jnp.int32, (128, 1), 0)
    for t in range(_CTX):
        gcopies[t].wait()
        a = idx_ref[t]
        oh = jnp.where(sub_iota == a % 128, 1.0, 0.0).astype(jnp.float32)
        erow = lax.dot_general(
            oh, gbufs[t % _NGB][...], (((0,), (1,)), ((), ())),
            preferred_element_type=jnp.float32,
        )
        if t + _NGB < _CTX:
            gcopies[t + _NGB] = gstart(t + _NGB)
        eflat[:, _D * t : _D * (t + 1)] = erow

    w1_copy.wait()
    h = jnp.dot(eflat[...], w1v[...], preferred_element_type=jnp.float32)
    h = jnp.maximum(h + b1_ref[...], 0.0)

    tail_copy.wait()
    zt = lax.dot_general(
        h, tailbuf[...], (((1,), (1,)), ((), ())),
        preferred_element_type=jnp.float32,
    )
    zt = zt + b2_ref[:, _TAIL_OFF:]
    out_ref[:, _TAIL_OFF:] = zt
    m = jnp.max(zt)
    s = jnp.sum(jnp.exp(zt - m))

    for j in range(_NCH):
        copies[j].wait()
        if j + _NBUF < _NCH:
            start(j + _NBUF)
        off = j * _BV
        z = lax.dot_general(
            h, bufs[j % _NBUF][...], (((1,), (1,)), ((), ())),
            preferred_element_type=jnp.float32,
        )
        z = z + b2_ref[:, off : off + _BV]
        out_ref[:, off : off + _BV] = z
        bm = jnp.max(z)
        mn = jnp.maximum(m, bm)
        s = s * jnp.exp(m - mn) + jnp.sum(jnp.exp(z - mn))
        m = mn

    lse = m + jnp.log(s)
    out_ref[...] = out_ref[...] - lse


def kernel(inputs, emb_table, W1, b1, W2, b2):
    tableT = emb_table.T
    w2t = W2.T

    log_probs = pl.pallas_call(
        _body,
        in_specs=[
            pl.BlockSpec(memory_space=pltpu.SMEM),
            pl.BlockSpec(memory_space=pltpu.VMEM),
            pl.BlockSpec(memory_space=pltpu.VMEM),
            pl.BlockSpec(memory_space=pltpu.MemorySpace.HBM),
            pl.BlockSpec(memory_space=pltpu.MemorySpace.HBM),
            pl.BlockSpec(memory_space=pltpu.MemorySpace.HBM),
        ],
        out_specs=pl.BlockSpec(memory_space=pltpu.VMEM),
        out_shape=jax.ShapeDtypeStruct((1, _CARDS), jnp.float32),
        scratch_shapes=(
            [
                pltpu.VMEM((_IN1, _HID), jnp.float32),
                pltpu.VMEM((_TAIL_W, _HID), jnp.float32),
                pltpu.VMEM((1, _IN1), jnp.float32),
            ]
            + [pltpu.VMEM((_BV, _HID), jnp.float32) for _ in range(_NBUF)]
            + [pltpu.VMEM((_D, 128), jnp.float32) for _ in range(_NGB)]
            + [pltpu.SemaphoreType.DMA for _ in range(_NBUF + _NGB + 2)]
        ),
    )(inputs, b1.reshape(1, _HID), b2.reshape(1, _CARDS), tableT, W1, w2t)

    return log_probs
